# baseline (speedup 1.0000x reference)
import jax
import jax.numpy as jnp
from jax import lax
from jax.experimental import pallas as pl
from jax.experimental.pallas import tpu as pltpu
from jax.experimental.pallas import tpu_sc as plsc

N_NODES = 10000
N_EDGES = 320000
NF = 128

NC = 2
NT = 16
K = 80
EPT = N_EDGES // (NC * NT)
CHUNKS = EPT // K
STRIPE = 624
TAIL = N_NODES - NT * STRIPE


def _matmul(x, W):
    def body(x_ref, w_ref, o_ref):
        o_ref[...] = jnp.dot(x_ref[...], w_ref[...],
                             preferred_element_type=jnp.float32)

    return pl.pallas_call(
        body,
        grid=(10,),
        in_specs=[
            pl.BlockSpec((N_NODES // 10, NF), lambda i: (i, 0)),
            pl.BlockSpec((NF, NF), lambda i: (0, 0)),
        ],
        out_specs=pl.BlockSpec((N_NODES // 10, NF), lambda i: (i, 0)),
        out_shape=jax.ShapeDtypeStruct((N_NODES, NF), jnp.float32),
    )(x, W)


def _deg_body(dst_hbm, out_hbm, di_all, ones_v, zeros_v, deg_sh):
    c = lax.axis_index("c")
    s = lax.axis_index("s")

    @pl.loop(0, K, step=16)
    def _(j):
        ones_v[pl.ds(j, 16)] = jnp.ones((16,), jnp.float32)

    @pl.when(s == 0)
    def _():
        @pl.loop(0, N_NODES, step=16)
        def _(j):
            zeros_v[pl.ds(j, 16)] = jnp.zeros((16,), jnp.float32)

        pltpu.sync_copy(zeros_v, deg_sh)

    plsc.subcore_barrier()

    pltpu.sync_copy(dst_hbm.at[c * NT + s], di_all)

    @pl.loop(0, CHUNKS)
    def _(j):
        pltpu.sync_copy(ones_v, deg_sh.at[di_all.at[j]], add=True)

    plsc.subcore_barrier()

    @pl.when(s == 0)
    def _():
        pltpu.sync_copy(deg_sh, zeros_v)
        pltpu.sync_copy(zeros_v, out_hbm.at[pl.ds(c * N_NODES, N_NODES)])


def _deg(dst3d):
    mesh = plsc.VectorSubcoreMesh(core_axis_name="c", subcore_axis_name="s")
    call = pl.kernel(
        _deg_body,
        out_type=jax.ShapeDtypeStruct((NC * N_NODES,), jnp.float32),
        mesh=mesh,
        scratch_types=[
            pltpu.VMEM((CHUNKS, K), jnp.int32),
            pltpu.VMEM((K,), jnp.float32),
            pltpu.VMEM((N_NODES,), jnp.float32),
            pltpu.VMEM_SHARED((N_NODES,), jnp.float32),
        ],
    )
    return call(dst3d)


def _scale(h, degp_t):
    def body(h_ref, d_ref, g_ref):
        deg = d_ref[:, 0:1] + d_ref[:, 1:2] + 1.0
        g_ref[...] = h_ref[...] * lax.rsqrt(deg)

    return pl.pallas_call(
        body,
        grid=(10,),
        in_specs=[
            pl.BlockSpec((N_NODES // 10, NF), lambda i: (i, 0)),
            pl.BlockSpec((N_NODES // 10, NC), lambda i: (i, 0)),
        ],
        out_specs=pl.BlockSpec((N_NODES // 10, NF), lambda i: (i, 0)),
        out_shape=jax.ShapeDtypeStruct((N_NODES, NF), jnp.float32),
    )(h, degp_t)


def _edge_body(g_hbm, src_hbm, dst_hbm, out_hbm,
               si_v, di_all, rows0, rows1, acc_sh,
               gsem0, gsem1, ssem0, ssem1):
    c = lax.axis_index("c")
    s = lax.axis_index("s")

    @pl.loop(0, K)
    def _(r):
        @pl.loop(0, NF, step=16)
        def _(j):
            rows0[r, pl.ds(j, 16)] = jnp.zeros((16,), jnp.float32)

    r0 = s * STRIPE

    @pl.loop(0, 560, step=K)
    def _(k):
        pltpu.sync_copy(rows0, acc_sh.at[pl.ds(r0 + k, K)])

    pltpu.sync_copy(rows0.at[pl.ds(0, STRIPE - 560)],
                    acc_sh.at[pl.ds(r0 + 560, STRIPE - 560)])

    @pl.when(s == NT - 1)
    def _():
        pltpu.sync_copy(rows0.at[pl.ds(0, TAIL)],
                        acc_sh.at[pl.ds(NT * STRIPE, TAIL)])

    plsc.subcore_barrier()

    pltpu.sync_copy(src_hbm.at[pl.ds((c * NT + s) * EPT, EPT)], si_v)
    pltpu.sync_copy(dst_hbm.at[c * NT + s], di_all)

    def g_start(buf, sem, j):
        pltpu.async_copy(g_hbm.at[si_v.at[pl.ds(j * K, K)]], buf, sem)

    def g_wait(buf, sem, j):
        pltpu.make_async_copy(g_hbm.at[si_v.at[pl.ds(j * K, K)]],
                              buf, sem).wait()

    def s_start(buf, sem, j):
        pltpu.async_copy(buf, acc_sh.at[di_all.at[j]], sem, add=True)

    def s_wait(buf, sem, j):
        pltpu.make_async_copy(buf, acc_sh.at[di_all.at[j]], sem).wait()

    g_start(rows0, gsem0, 0)
    g_start(rows1, gsem1, 1)

    @pl.loop(0, CHUNKS - 3, step=2)
    def _(j):
        g_wait(rows0, gsem0, j)
        s_start(rows0, ssem0, j)
        g_wait(rows1, gsem1, j + 1)
        s_start(rows1, ssem1, j + 1)
        s_wait(rows0, ssem0, j)
        g_start(rows0, gsem0, j + 2)
        s_wait(rows1, ssem1, j + 1)
        g_start(rows1, gsem1, j + 3)

    g_wait(rows0, gsem0, CHUNKS - 3)
    s_start(rows0, ssem0, CHUNKS - 3)
    g_wait(rows1, gsem1, CHUNKS - 2)
    s_start(rows1, ssem1, CHUNKS - 2)
    s_wait(rows0, ssem0, CHUNKS - 3)
    g_start(rows0, gsem0, CHUNKS - 1)
    g_wait(rows0, gsem0, CHUNKS - 1)
    s_start(rows0, ssem0, CHUNKS - 1)
    s_wait(rows1, ssem1, CHUNKS - 2)
    s_wait(rows0, ssem0, CHUNKS - 1)

    plsc.subcore_barrier()
    pltpu.sync_copy(acc_sh.at[pl.ds(r0, STRIPE)],
                    out_hbm.at[c, pl.ds(r0, STRIPE)])

    @pl.when(s == NT - 1)
    def _():
        pltpu.sync_copy(acc_sh.at[pl.ds(NT * STRIPE, TAIL)],
                        out_hbm.at[c, pl.ds(NT * STRIPE, TAIL)])


def _edge_pass(g, src_flat, dst3d):
    mesh = plsc.VectorSubcoreMesh(core_axis_name="c", subcore_axis_name="s")
    call = pl.kernel(
        _edge_body,
        out_type=jax.ShapeDtypeStruct((NC, N_NODES, NF), jnp.float32),
        mesh=mesh,
        scratch_types=[
            pltpu.VMEM((EPT,), jnp.int32),
            pltpu.VMEM((CHUNKS, K), jnp.int32),
            pltpu.VMEM((K, NF), jnp.float32),
            pltpu.VMEM((K, NF), jnp.float32),
            pltpu.VMEM_SHARED((N_NODES, NF), jnp.float32),
            pltpu.SemaphoreType.DMA,
            pltpu.SemaphoreType.DMA,
            pltpu.SemaphoreType.DMA,
            pltpu.SemaphoreType.DMA,
        ],
    )
    return call(g, src_flat, dst3d)


def _final(acc_p, h, degp_t, b2d):
    def body(a_ref, h_ref, d_ref, b_ref, o_ref):
        deg = d_ref[:, 0:1] + d_ref[:, 1:2] + 1.0
        dis = lax.rsqrt(deg)
        acc = a_ref[0].astype(jnp.float32) + a_ref[1].astype(jnp.float32)
        o_ref[...] = acc * dis + h_ref[...] * (1.0 / deg) + b_ref[...]

    return pl.pallas_call(
        body,
        grid=(10,),
        in_specs=[
            pl.BlockSpec((NC, N_NODES // 10, NF), lambda i: (0, i, 0)),
            pl.BlockSpec((N_NODES // 10, NF), lambda i: (i, 0)),
            pl.BlockSpec((N_NODES // 10, NC), lambda i: (i, 0)),
            pl.BlockSpec((1, NF), lambda i: (0, 0)),
        ],
        out_specs=pl.BlockSpec((N_NODES // 10, NF), lambda i: (i, 0)),
        out_shape=jax.ShapeDtypeStruct((N_NODES, NF), jnp.float32),
    )(acc_p, h, degp_t, b2d)


def kernel(x, edge_index, W, b):
    src_flat = edge_index[0].astype(jnp.int32).reshape(N_EDGES)
    dst3d = edge_index[1].astype(jnp.int32).reshape(NC * NT, CHUNKS, K)

    h = _matmul(x, W)
    deg_p = _deg(dst3d)
    degp_t = deg_p.reshape(NC, N_NODES).T
    g = _scale(h, degp_t)
    acc_p = _edge_pass(g, src_flat, dst3d)
    return _final(acc_p, h, degp_t, b.reshape(1, NF))


# speedup vs baseline: 1.0109x; 1.0109x over previous
import jax
import jax.numpy as jnp
from jax import lax
from jax.experimental import pallas as pl
from jax.experimental.pallas import tpu as pltpu
from jax.experimental.pallas import tpu_sc as plsc

N_NODES = 10000
N_EDGES = 320000
NF = 128

NC = 2
NT = 16
K = 80
EPT = N_EDGES // (NC * NT)
CHUNKS = EPT // K
STRIPE = 624
TAIL = N_NODES - NT * STRIPE


def _matmul(x, W):
    def body(x_ref, w_ref, o_ref):
        o_ref[...] = jnp.dot(x_ref[...], w_ref[...],
                             preferred_element_type=jnp.float32)

    return pl.pallas_call(
        body,
        grid=(10,),
        in_specs=[
            pl.BlockSpec((N_NODES // 10, NF), lambda i: (i, 0)),
            pl.BlockSpec((NF, NF), lambda i: (0, 0)),
        ],
        out_specs=pl.BlockSpec((N_NODES // 10, NF), lambda i: (i, 0)),
        out_shape=jax.ShapeDtypeStruct((N_NODES, NF), jnp.float32),
    )(x, W)


def _deg_body(dst_hbm, out_hbm, di_all, ones_v, zeros_v, deg_sh):
    c = lax.axis_index("c")
    s = lax.axis_index("s")

    @pl.loop(0, K, step=16)
    def _(j):
        ones_v[pl.ds(j, 16)] = jnp.ones((16,), jnp.float32)

    @pl.when(s == 0)
    def _():
        @pl.loop(0, N_NODES, step=16)
        def _(j):
            zeros_v[pl.ds(j, 16)] = jnp.zeros((16,), jnp.float32)

        pltpu.sync_copy(zeros_v, deg_sh)

    plsc.subcore_barrier()

    pltpu.sync_copy(dst_hbm.at[c * NT + s], di_all)

    @pl.loop(0, CHUNKS)
    def _(j):
        pltpu.sync_copy(ones_v, deg_sh.at[di_all.at[j]], add=True)

    plsc.subcore_barrier()

    @pl.when(s == 0)
    def _():
        pltpu.sync_copy(deg_sh, zeros_v)
        pltpu.sync_copy(zeros_v, out_hbm.at[pl.ds(c * N_NODES, N_NODES)])


def _deg(dst3d):
    mesh = plsc.VectorSubcoreMesh(core_axis_name="c", subcore_axis_name="s")
    call = pl.kernel(
        _deg_body,
        out_type=jax.ShapeDtypeStruct((NC * N_NODES,), jnp.float32),
        mesh=mesh,
        scratch_types=[
            pltpu.VMEM((CHUNKS, K), jnp.int32),
            pltpu.VMEM((K,), jnp.float32),
            pltpu.VMEM((N_NODES,), jnp.float32),
            pltpu.VMEM_SHARED((N_NODES,), jnp.float32),
        ],
    )
    return call(dst3d)


def _scale(h, degp_t):
    def body(h_ref, d_ref, g_ref):
        deg = d_ref[:, 0:1] + d_ref[:, 1:2] + 1.0
        g_ref[...] = h_ref[...] * lax.rsqrt(deg)

    return pl.pallas_call(
        body,
        grid=(10,),
        in_specs=[
            pl.BlockSpec((N_NODES // 10, NF), lambda i: (i, 0)),
            pl.BlockSpec((N_NODES // 10, NC), lambda i: (i, 0)),
        ],
        out_specs=pl.BlockSpec((N_NODES // 10, NF), lambda i: (i, 0)),
        out_shape=jax.ShapeDtypeStruct((N_NODES, NF), jnp.float32),
    )(h, degp_t)


def _edge_body(g_hbm, src_hbm, dst_hbm, out_hbm,
               si_v, di_all, rows0, rows1, acc_sh,
               gsem0, gsem1, ssem0, ssem1):
    c = lax.axis_index("c")
    s = lax.axis_index("s")

    pltpu.async_copy(src_hbm.at[pl.ds((c * NT + s) * EPT, EPT)], si_v, ssem0)
    pltpu.async_copy(dst_hbm.at[c * NT + s], di_all, ssem1)

    @pl.loop(0, K)
    def _(r):
        @pl.loop(0, NF, step=16)
        def _(j):
            rows0[r, pl.ds(j, 16)] = jnp.zeros((16,), jnp.float32)

    r0 = s * STRIPE

    @pl.loop(0, 560, step=K)
    def _(k):
        pltpu.sync_copy(rows0, acc_sh.at[pl.ds(r0 + k, K)])

    pltpu.sync_copy(rows0.at[pl.ds(0, STRIPE - 560)],
                    acc_sh.at[pl.ds(r0 + 560, STRIPE - 560)])

    @pl.when(s == NT - 1)
    def _():
        pltpu.sync_copy(rows0.at[pl.ds(0, TAIL)],
                        acc_sh.at[pl.ds(NT * STRIPE, TAIL)])

    pltpu.make_async_copy(src_hbm.at[pl.ds((c * NT + s) * EPT, EPT)],
                          si_v, ssem0).wait()
    pltpu.make_async_copy(dst_hbm.at[c * NT + s], di_all, ssem1).wait()

    def g_start(buf, sem, j):
        pltpu.async_copy(g_hbm.at[si_v.at[pl.ds(j * K, K)]], buf, sem)

    def g_wait(buf, sem, j):
        pltpu.make_async_copy(g_hbm.at[si_v.at[pl.ds(j * K, K)]],
                              buf, sem).wait()

    def s_start(buf, sem, j):
        pltpu.async_copy(buf, acc_sh.at[di_all.at[j]], sem, add=True)

    def s_wait(buf, sem, j):
        pltpu.make_async_copy(buf, acc_sh.at[di_all.at[j]], sem).wait()

    g_start(rows0, gsem0, 0)
    g_start(rows1, gsem1, 1)

    plsc.subcore_barrier()

    @pl.loop(0, CHUNKS - 3, step=2)
    def _(j):
        g_wait(rows0, gsem0, j)
        s_start(rows0, ssem0, j)
        g_wait(rows1, gsem1, j + 1)
        s_start(rows1, ssem1, j + 1)
        s_wait(rows0, ssem0, j)
        g_start(rows0, gsem0, j + 2)
        s_wait(rows1, ssem1, j + 1)
        g_start(rows1, gsem1, j + 3)

    g_wait(rows0, gsem0, CHUNKS - 3)
    s_start(rows0, ssem0, CHUNKS - 3)
    g_wait(rows1, gsem1, CHUNKS - 2)
    s_start(rows1, ssem1, CHUNKS - 2)
    s_wait(rows0, ssem0, CHUNKS - 3)
    g_start(rows0, gsem0, CHUNKS - 1)
    g_wait(rows0, gsem0, CHUNKS - 1)
    s_start(rows0, ssem0, CHUNKS - 1)
    s_wait(rows1, ssem1, CHUNKS - 2)
    s_wait(rows0, ssem0, CHUNKS - 1)

    plsc.subcore_barrier()
    pltpu.sync_copy(acc_sh.at[pl.ds(r0, STRIPE)],
                    out_hbm.at[c, pl.ds(r0, STRIPE)])

    @pl.when(s == NT - 1)
    def _():
        pltpu.sync_copy(acc_sh.at[pl.ds(NT * STRIPE, TAIL)],
                        out_hbm.at[c, pl.ds(NT * STRIPE, TAIL)])


def _edge_pass(g, src_flat, dst3d):
    mesh = plsc.VectorSubcoreMesh(core_axis_name="c", subcore_axis_name="s")
    call = pl.kernel(
        _edge_body,
        out_type=jax.ShapeDtypeStruct((NC, N_NODES, NF), jnp.float32),
        mesh=mesh,
        scratch_types=[
            pltpu.VMEM((EPT,), jnp.int32),
            pltpu.VMEM((CHUNKS, K), jnp.int32),
            pltpu.VMEM((K, NF), jnp.float32),
            pltpu.VMEM((K, NF), jnp.float32),
            pltpu.VMEM_SHARED((N_NODES, NF), jnp.float32),
            pltpu.SemaphoreType.DMA,
            pltpu.SemaphoreType.DMA,
            pltpu.SemaphoreType.DMA,
            pltpu.SemaphoreType.DMA,
        ],
    )
    return call(g, src_flat, dst3d)


def _final(acc_p, h, degp_t, b2d):
    def body(a_ref, h_ref, d_ref, b_ref, o_ref):
        deg = d_ref[:, 0:1] + d_ref[:, 1:2] + 1.0
        dis = lax.rsqrt(deg)
        acc = a_ref[0].astype(jnp.float32) + a_ref[1].astype(jnp.float32)
        o_ref[...] = acc * dis + h_ref[...] * (1.0 / deg) + b_ref[...]

    return pl.pallas_call(
        body,
        grid=(10,),
        in_specs=[
            pl.BlockSpec((NC, N_NODES // 10, NF), lambda i: (0, i, 0)),
            pl.BlockSpec((N_NODES // 10, NF), lambda i: (i, 0)),
            pl.BlockSpec((N_NODES // 10, NC), lambda i: (i, 0)),
            pl.BlockSpec((1, NF), lambda i: (0, 0)),
        ],
        out_specs=pl.BlockSpec((N_NODES // 10, NF), lambda i: (i, 0)),
        out_shape=jax.ShapeDtypeStruct((N_NODES, NF), jnp.float32),
    )(acc_p, h, degp_t, b2d)


def kernel(x, edge_index, W, b):
    src_flat = edge_index[0].astype(jnp.int32).reshape(N_EDGES)
    dst3d = edge_index[1].astype(jnp.int32).reshape(NC * NT, CHUNKS, K)

    h = _matmul(x, W)
    deg_p = _deg(dst3d)
    degp_t = deg_p.reshape(NC, N_NODES).T
    g = _scale(h, degp_t)
    acc_p = _edge_pass(g, src_flat, dst3d)
    return _final(acc_p, h, degp_t, b.reshape(1, NF))


# speedup vs baseline: 1.0518x; 1.0404x over previous
import jax
import jax.numpy as jnp
from jax import lax
from jax.experimental import pallas as pl
from jax.experimental.pallas import tpu as pltpu
from jax.experimental.pallas import tpu_sc as plsc

N_NODES = 10000
N_EDGES = 320000
NF = 128

NC = 2
NT = 16
K = 80
EPT = N_EDGES // (NC * NT)
CHUNKS = EPT // K
STRIPE = 624
TAIL = N_NODES - NT * STRIPE


def _matmul(x, W):
    def body(x_ref, w_ref, o_ref):
        o_ref[...] = jnp.dot(x_ref[...], w_ref[...],
                             preferred_element_type=jnp.float32)

    return pl.pallas_call(
        body,
        grid=(10,),
        in_specs=[
            pl.BlockSpec((N_NODES // 10, NF), lambda i: (i, 0)),
            pl.BlockSpec((NF, NF), lambda i: (0, 0)),
        ],
        out_specs=pl.BlockSpec((N_NODES // 10, NF), lambda i: (i, 0)),
        out_shape=jax.ShapeDtypeStruct((N_NODES, NF), jnp.float32),
    )(x, W)


def _deg_body(dst_hbm, out_hbm, di_all, ones_v, zeros_v, deg_sh, dsem, lsem):
    c = lax.axis_index("c")
    s = lax.axis_index("s")

    pltpu.async_copy(dst_hbm.at[c * NT + s], di_all, lsem)

    @pl.loop(0, K, step=16)
    def _(j):
        ones_v[pl.ds(j, 16)] = jnp.ones((16,), jnp.float32)

    @pl.when(s == 0)
    def _():
        @pl.loop(0, N_NODES, step=16)
        def _(j):
            zeros_v[pl.ds(j, 16)] = jnp.zeros((16,), jnp.float32)

        pltpu.sync_copy(zeros_v, deg_sh)

    pltpu.make_async_copy(dst_hbm.at[c * NT + s], di_all, lsem).wait()
    plsc.subcore_barrier()

    @pl.loop(0, CHUNKS)
    def _(j):
        pltpu.async_copy(ones_v, deg_sh.at[di_all.at[j]], dsem, add=True)

    @pl.loop(0, CHUNKS)
    def _(j):
        pltpu.make_async_copy(ones_v, deg_sh.at[di_all.at[j]], dsem).wait()

    plsc.subcore_barrier()

    @pl.when(s == 0)
    def _():
        pltpu.sync_copy(deg_sh, zeros_v)
        pltpu.sync_copy(zeros_v, out_hbm.at[pl.ds(c * N_NODES, N_NODES)])


def _deg(dst3d):
    mesh = plsc.VectorSubcoreMesh(core_axis_name="c", subcore_axis_name="s")
    call = pl.kernel(
        _deg_body,
        out_type=jax.ShapeDtypeStruct((NC * N_NODES,), jnp.float32),
        mesh=mesh,
        scratch_types=[
            pltpu.VMEM((CHUNKS, K), jnp.int32),
            pltpu.VMEM((K,), jnp.float32),
            pltpu.VMEM((N_NODES,), jnp.float32),
            pltpu.VMEM_SHARED((N_NODES,), jnp.float32),
            pltpu.SemaphoreType.DMA,
            pltpu.SemaphoreType.DMA,
        ],
    )
    return call(dst3d)


def _scale(h, degp_t):
    def body(h_ref, d_ref, g_ref):
        deg = d_ref[:, 0:1] + d_ref[:, 1:2] + 1.0
        g_ref[...] = h_ref[...] * lax.rsqrt(deg)

    return pl.pallas_call(
        body,
        grid=(10,),
        in_specs=[
            pl.BlockSpec((N_NODES // 10, NF), lambda i: (i, 0)),
            pl.BlockSpec((N_NODES // 10, NC), lambda i: (i, 0)),
        ],
        out_specs=pl.BlockSpec((N_NODES // 10, NF), lambda i: (i, 0)),
        out_shape=jax.ShapeDtypeStruct((N_NODES, NF), jnp.float32),
    )(h, degp_t)


def _edge_body(g_hbm, src_hbm, dst_hbm, out_hbm,
               si_v, di_all, rows0, rows1, acc_sh,
               gsem0, gsem1, ssem0, ssem1):
    c = lax.axis_index("c")
    s = lax.axis_index("s")

    pltpu.async_copy(src_hbm.at[pl.ds((c * NT + s) * EPT, EPT)], si_v, ssem0)
    pltpu.async_copy(dst_hbm.at[c * NT + s], di_all, ssem1)

    @pl.loop(0, K)
    def _(r):
        @pl.loop(0, NF, step=16)
        def _(j):
            rows0[r, pl.ds(j, 16)] = jnp.zeros((16,), jnp.float32)

    r0 = s * STRIPE

    @pl.loop(0, 560, step=K)
    def _(k):
        pltpu.sync_copy(rows0, acc_sh.at[pl.ds(r0 + k, K)])

    pltpu.sync_copy(rows0.at[pl.ds(0, STRIPE - 560)],
                    acc_sh.at[pl.ds(r0 + 560, STRIPE - 560)])

    @pl.when(s == NT - 1)
    def _():
        pltpu.sync_copy(rows0.at[pl.ds(0, TAIL)],
                        acc_sh.at[pl.ds(NT * STRIPE, TAIL)])

    pltpu.make_async_copy(src_hbm.at[pl.ds((c * NT + s) * EPT, EPT)],
                          si_v, ssem0).wait()
    pltpu.make_async_copy(dst_hbm.at[c * NT + s], di_all, ssem1).wait()

    def g_start(buf, sem, j):
        pltpu.async_copy(g_hbm.at[si_v.at[pl.ds(j * K, K)]], buf, sem)

    def g_wait(buf, sem, j):
        pltpu.make_async_copy(g_hbm.at[si_v.at[pl.ds(j * K, K)]],
                              buf, sem).wait()

    def s_start(buf, sem, j):
        pltpu.async_copy(buf, acc_sh.at[di_all.at[j]], sem, add=True)

    def s_wait(buf, sem, j):
        pltpu.make_async_copy(buf, acc_sh.at[di_all.at[j]], sem).wait()

    g_start(rows0, gsem0, 0)
    g_start(rows1, gsem1, 1)

    plsc.subcore_barrier()

    @pl.loop(0, CHUNKS - 3, step=2)
    def _(j):
        g_wait(rows0, gsem0, j)
        s_start(rows0, ssem0, j)
        g_wait(rows1, gsem1, j + 1)
        s_start(rows1, ssem1, j + 1)
        s_wait(rows0, ssem0, j)
        g_start(rows0, gsem0, j + 2)
        s_wait(rows1, ssem1, j + 1)
        g_start(rows1, gsem1, j + 3)

    g_wait(rows0, gsem0, CHUNKS - 3)
    s_start(rows0, ssem0, CHUNKS - 3)
    g_wait(rows1, gsem1, CHUNKS - 2)
    s_start(rows1, ssem1, CHUNKS - 2)
    s_wait(rows0, ssem0, CHUNKS - 3)
    g_start(rows0, gsem0, CHUNKS - 1)
    g_wait(rows0, gsem0, CHUNKS - 1)
    s_start(rows0, ssem0, CHUNKS - 1)
    s_wait(rows1, ssem1, CHUNKS - 2)
    s_wait(rows0, ssem0, CHUNKS - 1)

    plsc.subcore_barrier()
    pltpu.sync_copy(acc_sh.at[pl.ds(r0, STRIPE)],
                    out_hbm.at[c, pl.ds(r0, STRIPE)])

    @pl.when(s == NT - 1)
    def _():
        pltpu.sync_copy(acc_sh.at[pl.ds(NT * STRIPE, TAIL)],
                        out_hbm.at[c, pl.ds(NT * STRIPE, TAIL)])


def _edge_pass(g, src_flat, dst3d):
    mesh = plsc.VectorSubcoreMesh(core_axis_name="c", subcore_axis_name="s")
    call = pl.kernel(
        _edge_body,
        out_type=jax.ShapeDtypeStruct((NC, N_NODES, NF), jnp.float32),
        mesh=mesh,
        scratch_types=[
            pltpu.VMEM((EPT,), jnp.int32),
            pltpu.VMEM((CHUNKS, K), jnp.int32),
            pltpu.VMEM((K, NF), jnp.float32),
            pltpu.VMEM((K, NF), jnp.float32),
            pltpu.VMEM_SHARED((N_NODES, NF), jnp.float32),
            pltpu.SemaphoreType.DMA,
            pltpu.SemaphoreType.DMA,
            pltpu.SemaphoreType.DMA,
            pltpu.SemaphoreType.DMA,
        ],
    )
    return call(g, src_flat, dst3d)


def _final(acc_p, h, degp_t, b2d):
    def body(a_ref, h_ref, d_ref, b_ref, o_ref):
        deg = d_ref[:, 0:1] + d_ref[:, 1:2] + 1.0
        dis = lax.rsqrt(deg)
        acc = a_ref[0].astype(jnp.float32) + a_ref[1].astype(jnp.float32)
        o_ref[...] = acc * dis + h_ref[...] * (1.0 / deg) + b_ref[...]

    return pl.pallas_call(
        body,
        grid=(10,),
        in_specs=[
            pl.BlockSpec((NC, N_NODES // 10, NF), lambda i: (0, i, 0)),
            pl.BlockSpec((N_NODES // 10, NF), lambda i: (i, 0)),
            pl.BlockSpec((N_NODES // 10, NC), lambda i: (i, 0)),
            pl.BlockSpec((1, NF), lambda i: (0, 0)),
        ],
        out_specs=pl.BlockSpec((N_NODES // 10, NF), lambda i: (i, 0)),
        out_shape=jax.ShapeDtypeStruct((N_NODES, NF), jnp.float32),
    )(acc_p, h, degp_t, b2d)


def kernel(x, edge_index, W, b):
    src_flat = edge_index[0].astype(jnp.int32).reshape(N_EDGES)
    dst3d = edge_index[1].astype(jnp.int32).reshape(NC * NT, CHUNKS, K)

    h = _matmul(x, W)
    deg_p = _deg(dst3d)
    degp_t = deg_p.reshape(NC, N_NODES).T
    g = _scale(h, degp_t)
    acc_p = _edge_pass(g, src_flat, dst3d)
    return _final(acc_p, h, degp_t, b.reshape(1, NF))


# speedup vs baseline: 1.0565x; 1.0044x over previous
import jax
import jax.numpy as jnp
from jax import lax
from jax.experimental import pallas as pl
from jax.experimental.pallas import tpu as pltpu
from jax.experimental.pallas import tpu_sc as plsc

N_NODES = 10000
N_EDGES = 320000
NF = 128

NC = 2
NT = 16
K = 80
EPT = N_EDGES // (NC * NT)
CHUNKS = EPT // K
STRIPE = 624
TAIL = N_NODES - NT * STRIPE


def _mm_scale(x, W, degp_t):
    def body(x_ref, w_ref, d_ref, g_ref):
        deg = d_ref[:, 0:1] + d_ref[:, 1:2] + 1.0
        xs = x_ref[...] * lax.rsqrt(deg)
        g_ref[...] = jnp.dot(xs, w_ref[...],
                             preferred_element_type=jnp.float32)

    return pl.pallas_call(
        body,
        grid=(10,),
        in_specs=[
            pl.BlockSpec((N_NODES // 10, NF), lambda i: (i, 0)),
            pl.BlockSpec((NF, NF), lambda i: (0, 0)),
            pl.BlockSpec((N_NODES // 10, NC), lambda i: (i, 0)),
        ],
        out_specs=pl.BlockSpec((N_NODES // 10, NF), lambda i: (i, 0)),
        out_shape=jax.ShapeDtypeStruct((N_NODES, NF), jnp.float32),
    )(x, W, degp_t)


def _deg_body(dst_hbm, out_hbm, di_all, ones_v, zeros_v, deg_sh, dsem, lsem):
    c = lax.axis_index("c")
    s = lax.axis_index("s")

    pltpu.async_copy(dst_hbm.at[c * NT + s], di_all, lsem)

    @pl.loop(0, K, step=16)
    def _(j):
        ones_v[pl.ds(j, 16)] = jnp.ones((16,), jnp.float32)

    @pl.when(s == 0)
    def _():
        @pl.loop(0, N_NODES, step=16)
        def _(j):
            zeros_v[pl.ds(j, 16)] = jnp.zeros((16,), jnp.float32)

        pltpu.sync_copy(zeros_v, deg_sh)

    pltpu.make_async_copy(dst_hbm.at[c * NT + s], di_all, lsem).wait()
    plsc.subcore_barrier()

    @pl.loop(0, CHUNKS)
    def _(j):
        pltpu.async_copy(ones_v, deg_sh.at[di_all.at[j]], dsem, add=True)

    @pl.loop(0, CHUNKS)
    def _(j):
        pltpu.make_async_copy(ones_v, deg_sh.at[di_all.at[j]], dsem).wait()

    plsc.subcore_barrier()

    @pl.when(s == 0)
    def _():
        pltpu.sync_copy(deg_sh, zeros_v)
        pltpu.sync_copy(zeros_v, out_hbm.at[pl.ds(c * N_NODES, N_NODES)])


def _deg(dst3d):
    mesh = plsc.VectorSubcoreMesh(core_axis_name="c", subcore_axis_name="s")
    call = pl.kernel(
        _deg_body,
        out_type=jax.ShapeDtypeStruct((NC * N_NODES,), jnp.float32),
        mesh=mesh,
        scratch_types=[
            pltpu.VMEM((CHUNKS, K), jnp.int32),
            pltpu.VMEM((K,), jnp.float32),
            pltpu.VMEM((N_NODES,), jnp.float32),
            pltpu.VMEM_SHARED((N_NODES,), jnp.float32),
            pltpu.SemaphoreType.DMA,
            pltpu.SemaphoreType.DMA,
        ],
    )
    return call(dst3d)


def _edge_body(g_hbm, src_hbm, dst_hbm, out_hbm,
               si_v, di_all, rows0, rows1, acc_sh,
               gsem0, gsem1, ssem0, ssem1):
    c = lax.axis_index("c")
    s = lax.axis_index("s")

    pltpu.async_copy(src_hbm.at[pl.ds((c * NT + s) * EPT, EPT)], si_v, ssem0)
    pltpu.async_copy(dst_hbm.at[c * NT + s], di_all, ssem1)

    @pl.loop(0, K)
    def _(r):
        @pl.loop(0, NF, step=16)
        def _(j):
            rows0[r, pl.ds(j, 16)] = jnp.zeros((16,), jnp.float32)

    r0 = s * STRIPE

    @pl.loop(0, 560, step=K)
    def _(k):
        pltpu.sync_copy(rows0, acc_sh.at[pl.ds(r0 + k, K)])

    pltpu.sync_copy(rows0.at[pl.ds(0, STRIPE - 560)],
                    acc_sh.at[pl.ds(r0 + 560, STRIPE - 560)])

    @pl.when(s == NT - 1)
    def _():
        pltpu.sync_copy(rows0.at[pl.ds(0, TAIL)],
                        acc_sh.at[pl.ds(NT * STRIPE, TAIL)])

    pltpu.make_async_copy(src_hbm.at[pl.ds((c * NT + s) * EPT, EPT)],
                          si_v, ssem0).wait()
    pltpu.make_async_copy(dst_hbm.at[c * NT + s], di_all, ssem1).wait()

    def g_start(buf, sem, j):
        pltpu.async_copy(g_hbm.at[si_v.at[pl.ds(j * K, K)]], buf, sem)

    def g_wait(buf, sem, j):
        pltpu.make_async_copy(g_hbm.at[si_v.at[pl.ds(j * K, K)]],
                              buf, sem).wait()

    def s_start(buf, sem, j):
        pltpu.async_copy(buf, acc_sh.at[di_all.at[j]], sem, add=True)

    def s_wait(buf, sem, j):
        pltpu.make_async_copy(buf, acc_sh.at[di_all.at[j]], sem).wait()

    g_start(rows0, gsem0, 0)
    g_start(rows1, gsem1, 1)

    plsc.subcore_barrier()

    @pl.loop(0, CHUNKS - 3, step=2)
    def _(j):
        g_wait(rows0, gsem0, j)
        s_start(rows0, ssem0, j)
        g_wait(rows1, gsem1, j + 1)
        s_start(rows1, ssem1, j + 1)
        s_wait(rows0, ssem0, j)
        g_start(rows0, gsem0, j + 2)
        s_wait(rows1, ssem1, j + 1)
        g_start(rows1, gsem1, j + 3)

    g_wait(rows0, gsem0, CHUNKS - 3)
    s_start(rows0, ssem0, CHUNKS - 3)
    g_wait(rows1, gsem1, CHUNKS - 2)
    s_start(rows1, ssem1, CHUNKS - 2)
    s_wait(rows0, ssem0, CHUNKS - 3)
    g_start(rows0, gsem0, CHUNKS - 1)
    g_wait(rows0, gsem0, CHUNKS - 1)
    s_start(rows0, ssem0, CHUNKS - 1)
    s_wait(rows1, ssem1, CHUNKS - 2)
    s_wait(rows0, ssem0, CHUNKS - 1)

    plsc.subcore_barrier()
    pltpu.sync_copy(acc_sh.at[pl.ds(r0, STRIPE)],
                    out_hbm.at[c, pl.ds(r0, STRIPE)])

    @pl.when(s == NT - 1)
    def _():
        pltpu.sync_copy(acc_sh.at[pl.ds(NT * STRIPE, TAIL)],
                        out_hbm.at[c, pl.ds(NT * STRIPE, TAIL)])


def _edge_pass(g, src_flat, dst3d):
    mesh = plsc.VectorSubcoreMesh(core_axis_name="c", subcore_axis_name="s")
    call = pl.kernel(
        _edge_body,
        out_type=jax.ShapeDtypeStruct((NC, N_NODES, NF), jnp.float32),
        mesh=mesh,
        scratch_types=[
            pltpu.VMEM((EPT,), jnp.int32),
            pltpu.VMEM((CHUNKS, K), jnp.int32),
            pltpu.VMEM((K, NF), jnp.float32),
            pltpu.VMEM((K, NF), jnp.float32),
            pltpu.VMEM_SHARED((N_NODES, NF), jnp.float32),
            pltpu.SemaphoreType.DMA,
            pltpu.SemaphoreType.DMA,
            pltpu.SemaphoreType.DMA,
            pltpu.SemaphoreType.DMA,
        ],
    )
    return call(g, src_flat, dst3d)


def _final(acc_p, g, degp_t, b2d):
    def body(a_ref, g_ref, d_ref, b_ref, o_ref):
        deg = d_ref[:, 0:1] + d_ref[:, 1:2] + 1.0
        dis = lax.rsqrt(deg)
        acc = a_ref[0] + a_ref[1] + g_ref[...]
        o_ref[...] = acc * dis + b_ref[...]

    return pl.pallas_call(
        body,
        grid=(10,),
        in_specs=[
            pl.BlockSpec((NC, N_NODES // 10, NF), lambda i: (0, i, 0)),
            pl.BlockSpec((N_NODES // 10, NF), lambda i: (i, 0)),
            pl.BlockSpec((N_NODES // 10, NC), lambda i: (i, 0)),
            pl.BlockSpec((1, NF), lambda i: (0, 0)),
        ],
        out_specs=pl.BlockSpec((N_NODES // 10, NF), lambda i: (i, 0)),
        out_shape=jax.ShapeDtypeStruct((N_NODES, NF), jnp.float32),
    )(acc_p, g, degp_t, b2d)


def kernel(x, edge_index, W, b):
    src_flat = edge_index[0].astype(jnp.int32).reshape(N_EDGES)
    dst3d = edge_index[1].astype(jnp.int32).reshape(NC * NT, CHUNKS, K)

    deg_p = _deg(dst3d)
    degp_t = deg_p.reshape(NC, N_NODES).T
    g = _mm_scale(x, W, degp_t)
    acc_p = _edge_pass(g, src_flat, dst3d)
    return _final(acc_p, g, degp_t, b.reshape(1, NF))
